# baseline (speedup 1.0000x reference)
import functools

import jax
import jax.numpy as jnp
from jax import lax
from jax.experimental import pallas as pl
from jax.experimental.pallas import tpu as pltpu
from jax.experimental.pallas import tpu_sc as plsc

B, L, D = 4096, 200, 64
NW = 32
BW = B // NW
NBUF = 8
LOOKAHEAD = 6
NG = L // NBUF

_mesh = plsc.VectorSubcoreMesh(core_axis_name="c", subcore_axis_name="s")


@functools.partial(
    pl.kernel,
    mesh=_mesh,
    out_type=jax.ShapeDtypeStruct((L // 2, B, 2 * D), jnp.float32),
    scratch_types=[
        pltpu.VMEM((L, BW), jnp.int32),
        pltpu.VMEM((NBUF, BW, D), jnp.float32),
        [pltpu.SemaphoreType.DMA] * NBUF,
        [pltpu.SemaphoreType.DMA] * NBUF,
    ],
    compiler_params=pltpu.CompilerParams(use_tc_tiling_on_sc=False, needs_layout_passes=False),
)
def _emb_gather(xt_hbm, table_hbm, out_hbm, idx_v, gbuf, gsems, ssems):
    wid = lax.axis_index("s") * 2 + lax.axis_index("c")
    pltpu.sync_copy(xt_hbm.at[:, pl.ds(wid * BW, BW)], idx_v)

    def gather_start(l, b):
        pltpu.make_async_copy(
            table_hbm.at[idx_v.at[l]], gbuf.at[b], gsems[b]
        ).start()

    def gather_wait(b):
        pltpu.make_async_copy(
            table_hbm.at[idx_v.at[0]], gbuf.at[b], gsems[b]
        ).wait()

    def out_slice(l):
        return out_hbm.at[
            lax.shift_right_logical(l, 1),
            pl.ds(wid * BW, BW),
            pl.ds(lax.bitwise_and(l, 1) * D, D),
        ]

    def store_start(l, b):
        pltpu.make_async_copy(gbuf.at[b], out_slice(l), ssems[b]).start()

    def store_wait(b):
        pltpu.make_async_copy(gbuf.at[b], out_slice(0), ssems[b]).wait()

    for b in range(LOOKAHEAD):
        gather_start(b, b)

    def body(g, carry):
        for b in range(NBUF):
            l = g * NBUF + b
            s = (b + LOOKAHEAD) % NBUF
            nl = l + LOOKAHEAD

            @pl.when(nl < L)
            def _():
                @pl.when(nl >= NBUF)
                def _():
                    store_wait(s)

                gather_start(nl, s)

            gather_wait(b)
            store_start(l, b)
        return carry

    lax.fori_loop(0, NG, body, 0)
    for b in range(NBUF):
        store_wait(b)


def _tc_transpose_body(in_ref, out_ref):
    blk = in_ref[0]
    out_ref[:, :, 0, :, :] = blk.T.reshape(2, D // 8, 8, BW)


_tc_transpose = pl.pallas_call(
    _tc_transpose_body,
    grid=(L // 2, B // BW),
    in_specs=[pl.BlockSpec((1, BW, 2 * D), lambda q, j: (q, j, 0))],
    out_specs=pl.BlockSpec((2, D // 8, 1, 8, BW), lambda q, j: (q, 0, j, 0, 0)),
    out_shape=jax.ShapeDtypeStruct((L, D // 8, B // BW, 8, BW), jnp.float32),
)


def kernel(x, table):
    xt = jnp.swapaxes(x, 0, 1)
    interm = _emb_gather(xt, table)
    p = _tc_transpose(interm)
    return p.transpose((2, 4, 0, 1, 3)).reshape(B, L, D)


# speedup vs baseline: 3.9006x; 3.9006x over previous
import functools

import jax
import jax.numpy as jnp
from jax import lax
from jax.experimental import pallas as pl
from jax.experimental.pallas import tpu as pltpu
from jax.experimental.pallas import tpu_sc as plsc

B, L, D = 4096, 200, 64
NW = 32
BW = B // NW
NBUF = 8
LOOKAHEAD = 6
NG = L // NBUF

_mesh = plsc.VectorSubcoreMesh(core_axis_name="c", subcore_axis_name="s")


@functools.partial(
    pl.kernel,
    mesh=_mesh,
    out_type=jax.ShapeDtypeStruct((L // 2, B, 2 * D), jnp.float32),
    scratch_types=[
        pltpu.VMEM((L, BW), jnp.int32),
        pltpu.VMEM((NBUF, BW, D), jnp.float32),
        [pltpu.SemaphoreType.DMA] * NBUF,
        [pltpu.SemaphoreType.DMA] * NBUF,
    ],
    compiler_params=pltpu.CompilerParams(use_tc_tiling_on_sc=False, needs_layout_passes=False),
)
def _emb_gather(xt_hbm, table_hbm, out_hbm, idx_v, gbuf, gsems, ssems):
    wid = lax.axis_index("s") * 2 + lax.axis_index("c")
    pltpu.sync_copy(xt_hbm.at[:, pl.ds(wid * BW, BW)], idx_v)

    def gather_start(l, b):
        pltpu.make_async_copy(
            table_hbm.at[idx_v.at[l]], gbuf.at[b], gsems[b]
        ).start()

    def gather_wait(b):
        pltpu.make_async_copy(
            table_hbm.at[idx_v.at[0]], gbuf.at[b], gsems[b]
        ).wait()

    def out_slice(l):
        return out_hbm.at[
            lax.shift_right_logical(l, 1),
            pl.ds(wid * BW, BW),
            pl.ds(lax.bitwise_and(l, 1) * D, D),
        ]

    def store_start(l, b):
        pltpu.make_async_copy(gbuf.at[b], out_slice(l), ssems[b]).start()

    def store_wait(b):
        pltpu.make_async_copy(gbuf.at[b], out_slice(0), ssems[b]).wait()

    for b in range(LOOKAHEAD):
        gather_start(b, b)

    def body(g, carry):
        for b in range(NBUF):
            l = g * NBUF + b
            s = (b + LOOKAHEAD) % NBUF
            nl = l + LOOKAHEAD

            @pl.when(nl < L)
            def _():
                @pl.when(nl >= NBUF)
                def _():
                    store_wait(s)

                gather_start(nl, s)

            gather_wait(b)
            store_start(l, b)
        return carry

    lax.fori_loop(0, NG, body, 0)
    for b in range(NBUF):
        store_wait(b)


QB = 10


def _tc_transpose_body(in_ref, out_ref):
    for i in range(QB):
        out_ref[2 * i : 2 * i + 2, :, 0, :, :] = in_ref[i].T.reshape(2, D // 8, 8, BW)


_tc_transpose = pl.pallas_call(
    _tc_transpose_body,
    grid=(L // 2 // QB, B // BW),
    in_specs=[pl.BlockSpec((QB, BW, 2 * D), lambda q, j: (q, j, 0))],
    out_specs=pl.BlockSpec((2 * QB, D // 8, 1, 8, BW), lambda q, j: (q, 0, j, 0, 0)),
    out_shape=jax.ShapeDtypeStruct((L, D // 8, B // BW, 8, BW), jnp.float32),
)


def kernel(x, table):
    xt = jnp.swapaxes(x, 0, 1)
    interm = _emb_gather(xt, table)
    p = _tc_transpose(interm)
    return p.transpose((2, 4, 0, 1, 3)).reshape(B, L, D)


# speedup vs baseline: 4.8433x; 1.2417x over previous
import functools

import jax
import jax.numpy as jnp
from jax import lax
from jax.experimental import pallas as pl
from jax.experimental.pallas import tpu as pltpu
from jax.experimental.pallas import tpu_sc as plsc

B, L, D = 4096, 200, 64
NW = 32
BW = B // NW
NBUF = 8
LOOKAHEAD = 6
NG = L // NBUF

_mesh = plsc.VectorSubcoreMesh(core_axis_name="c", subcore_axis_name="s")


@functools.partial(
    pl.kernel,
    mesh=_mesh,
    out_type=jax.ShapeDtypeStruct((L // 2, B, 2 * D), jnp.float32),
    scratch_types=[
        pltpu.VMEM((L, BW), jnp.int32),
        pltpu.VMEM((NBUF, BW, D), jnp.float32),
        [pltpu.SemaphoreType.DMA] * NBUF,
        [pltpu.SemaphoreType.DMA] * NBUF,
    ],
    compiler_params=pltpu.CompilerParams(use_tc_tiling_on_sc=False, needs_layout_passes=False),
)
def _emb_gather(xt_hbm, table_hbm, out_hbm, idx_v, gbuf, gsems, ssems):
    wid = lax.axis_index("s") * 2 + lax.axis_index("c")
    pltpu.sync_copy(xt_hbm.at[:, pl.ds(wid * BW, BW)], idx_v)

    def gather_start(l, b):
        pltpu.make_async_copy(
            table_hbm.at[idx_v.at[l]], gbuf.at[b], gsems[b]
        ).start()

    def gather_wait(b):
        pltpu.make_async_copy(
            table_hbm.at[idx_v.at[0]], gbuf.at[b], gsems[b]
        ).wait()

    def out_slice(l):
        return out_hbm.at[
            lax.shift_right_logical(l, 1),
            pl.ds(wid * BW, BW),
            pl.ds(lax.bitwise_and(l, 1) * D, D),
        ]

    def store_start(l, b):
        pltpu.make_async_copy(gbuf.at[b], out_slice(l), ssems[b]).start()

    def store_wait(b):
        pltpu.make_async_copy(gbuf.at[b], out_slice(0), ssems[b]).wait()

    for b in range(LOOKAHEAD):
        gather_start(b, b)

    def body(g, carry):
        for b in range(NBUF):
            l = g * NBUF + b
            s = (b + LOOKAHEAD) % NBUF
            nl = l + LOOKAHEAD

            @pl.when(nl < L)
            def _():
                @pl.when(nl >= NBUF)
                def _():
                    store_wait(s)

                gather_start(nl, s)

            gather_wait(b)
            store_start(l, b)
        return carry

    lax.fori_loop(0, NG, body, 0)
    for b in range(NBUF):
        store_wait(b)


QB = 25


def _tc_transpose_body(in_ref, out_ref):
    for i in range(QB):
        out_ref[2 * i : 2 * i + 2, :, 0, :, :] = in_ref[i].T.reshape(2, D // 8, 8, BW)


_tc_transpose = pl.pallas_call(
    _tc_transpose_body,
    grid=(L // 2 // QB, B // BW),
    in_specs=[pl.BlockSpec((QB, BW, 2 * D), lambda q, j: (q, j, 0))],
    out_specs=pl.BlockSpec((2 * QB, D // 8, 1, 8, BW), lambda q, j: (q, 0, j, 0, 0)),
    out_shape=jax.ShapeDtypeStruct((L, D // 8, B // BW, 8, BW), jnp.float32),
)


def kernel(x, table):
    xt = jnp.swapaxes(x, 0, 1)
    interm = _emb_gather(xt, table)
    p = _tc_transpose(interm)
    return p.transpose((2, 4, 0, 1, 3)).reshape(B, L, D)


# speedup vs baseline: 5.3396x; 1.1025x over previous
import functools

import jax
import jax.numpy as jnp
from jax import lax
from jax.experimental import pallas as pl
from jax.experimental.pallas import tpu as pltpu
from jax.experimental.pallas import tpu_sc as plsc

B, L, D = 4096, 200, 64
NW = 32
BW = B // NW
NBUF = 8
LOOKAHEAD = 6
NG = L // NBUF

_mesh = plsc.VectorSubcoreMesh(core_axis_name="c", subcore_axis_name="s")


@functools.partial(
    pl.kernel,
    mesh=_mesh,
    out_type=jax.ShapeDtypeStruct((L // 2, B, 2 * D), jnp.float32),
    scratch_types=[
        pltpu.VMEM((L, BW), jnp.int32),
        pltpu.VMEM((NBUF, BW, D), jnp.float32),
        [pltpu.SemaphoreType.DMA] * NBUF,
        [pltpu.SemaphoreType.DMA] * NBUF,
    ],
    compiler_params=pltpu.CompilerParams(use_tc_tiling_on_sc=False, needs_layout_passes=False),
)
def _emb_gather(xt_hbm, table_hbm, out_hbm, idx_v, gbuf, gsems, ssems):
    wid = lax.axis_index("s") * 2 + lax.axis_index("c")
    pltpu.sync_copy(xt_hbm.at[:, pl.ds(wid * BW, BW)], idx_v)

    def gather_start(l, b):
        pltpu.make_async_copy(
            table_hbm.at[idx_v.at[l]], gbuf.at[b], gsems[b]
        ).start()

    def gather_wait(b):
        pltpu.make_async_copy(
            table_hbm.at[idx_v.at[0]], gbuf.at[b], gsems[b]
        ).wait()

    def out_slice(l):
        return out_hbm.at[
            lax.shift_right_logical(l, 1),
            pl.ds(wid * BW, BW),
            pl.ds(lax.bitwise_and(l, 1) * D, D),
        ]

    def store_start(l, b):
        pltpu.make_async_copy(gbuf.at[b], out_slice(l), ssems[b]).start()

    def store_wait(b):
        pltpu.make_async_copy(gbuf.at[b], out_slice(0), ssems[b]).wait()

    for b in range(LOOKAHEAD):
        gather_start(b, b)

    def body(g, carry):
        for b in range(NBUF):
            l = g * NBUF + b
            s = (b + LOOKAHEAD) % NBUF
            nl = l + LOOKAHEAD

            @pl.when(nl < L)
            def _():
                @pl.when(nl >= NBUF)
                def _():
                    store_wait(s)

                gather_start(nl, s)

            gather_wait(b)
            store_start(l, b)
        return carry

    lax.fori_loop(0, NG, body, 0)
    for b in range(NBUF):
        store_wait(b)


QB = 50


def _tc_transpose_body(in_ref, out_ref):
    for i in range(QB):
        out_ref[2 * i : 2 * i + 2, :, 0, :, :] = in_ref[i].T.reshape(2, D // 8, 8, BW)


_tc_transpose = pl.pallas_call(
    _tc_transpose_body,
    grid=(L // 2 // QB, B // BW),
    in_specs=[pl.BlockSpec((QB, BW, 2 * D), lambda q, j: (q, j, 0))],
    out_specs=pl.BlockSpec((2 * QB, D // 8, 1, 8, BW), lambda q, j: (q, 0, j, 0, 0)),
    out_shape=jax.ShapeDtypeStruct((L, D // 8, B // BW, 8, BW), jnp.float32),
)


def kernel(x, table):
    xt = jnp.swapaxes(x, 0, 1)
    interm = _emb_gather(xt, table)
    p = _tc_transpose(interm)
    return p.transpose((2, 4, 0, 1, 3)).reshape(B, L, D)


# speedup vs baseline: 5.4412x; 1.0190x over previous
import functools

import jax
import jax.numpy as jnp
from jax import lax
from jax.experimental import pallas as pl
from jax.experimental.pallas import tpu as pltpu
from jax.experimental.pallas import tpu_sc as plsc

B, L, D = 4096, 200, 64
NW = 32
BW = B // NW
NBUF = 8
LOOKAHEAD = 6
NG = L // NBUF

_mesh = plsc.VectorSubcoreMesh(core_axis_name="c", subcore_axis_name="s")


@functools.partial(
    pl.kernel,
    mesh=_mesh,
    out_type=jax.ShapeDtypeStruct((L // 2, B, 2 * D), jnp.float32),
    scratch_types=[
        pltpu.VMEM((L, BW), jnp.int32),
        pltpu.VMEM((NBUF, BW, D), jnp.float32),
        [pltpu.SemaphoreType.DMA] * NBUF,
        [pltpu.SemaphoreType.DMA] * NBUF,
    ],
    compiler_params=pltpu.CompilerParams(use_tc_tiling_on_sc=False, needs_layout_passes=False),
)
def _emb_gather(xt_hbm, table_hbm, out_hbm, idx_v, gbuf, gsems, ssems):
    wid = lax.axis_index("s") * 2 + lax.axis_index("c")
    pltpu.sync_copy(xt_hbm.at[:, pl.ds(wid * BW, BW)], idx_v)

    def gather_start(l, b):
        pltpu.make_async_copy(
            table_hbm.at[idx_v.at[l]], gbuf.at[b], gsems[b]
        ).start()

    def gather_wait(b):
        pltpu.make_async_copy(
            table_hbm.at[idx_v.at[0]], gbuf.at[b], gsems[b]
        ).wait()

    def out_slice(l):
        return out_hbm.at[
            lax.shift_right_logical(l, 1),
            pl.ds(wid * BW, BW),
            pl.ds(lax.bitwise_and(l, 1) * D, D),
        ]

    def store_start(l, b):
        pltpu.make_async_copy(gbuf.at[b], out_slice(l), ssems[b]).start()

    def store_wait(b):
        pltpu.make_async_copy(gbuf.at[b], out_slice(0), ssems[b]).wait()

    for b in range(LOOKAHEAD):
        gather_start(b, b)

    def body(g, carry):
        for b in range(NBUF):
            l = g * NBUF + b
            s = (b + LOOKAHEAD) % NBUF
            nl = l + LOOKAHEAD

            @pl.when(nl < L)
            def _():
                @pl.when(nl >= NBUF)
                def _():
                    store_wait(s)

                gather_start(nl, s)

            gather_wait(b)
            store_start(l, b)
        return carry

    lax.fori_loop(0, NG, body, 0)
    for b in range(NBUF):
        store_wait(b)


QB = 100


def _tc_transpose_body(in_ref, out_ref):
    for i in range(QB):
        out_ref[2 * i : 2 * i + 2, :, 0, :, :] = in_ref[i].T.reshape(2, D // 8, 8, BW)


_tc_transpose = pl.pallas_call(
    _tc_transpose_body,
    grid=(L // 2 // QB, B // BW),
    in_specs=[pl.BlockSpec((QB, BW, 2 * D), lambda q, j: (q, j, 0))],
    out_specs=pl.BlockSpec((2 * QB, D // 8, 1, 8, BW), lambda q, j: (q, 0, j, 0, 0)),
    out_shape=jax.ShapeDtypeStruct((L, D // 8, B // BW, 8, BW), jnp.float32),
)


def kernel(x, table):
    xt = jnp.swapaxes(x, 0, 1)
    interm = _emb_gather(xt, table)
    p = _tc_transpose(interm)
    return p.transpose((2, 4, 0, 1, 3)).reshape(B, L, D)
